# baseline (speedup 1.0000x reference)
import functools

import jax
import jax.numpy as jnp
from jax import lax
from jax.experimental import pallas as pl
from jax.experimental.pallas import tpu as pltpu
from jax.experimental.pallas import tpu_sc as plsc

IMG = 14
P = IMG * IMG
D = 512
B = 8
TAU = 0.5 / P
BETA = 4.0
NC = 2
NS = 16
L = 16
NW = NC * NS
CPW = D // NW


def _sc_body(x_hbm, out_hbm, xv, outv, zv):
    wid = lax.axis_index("s") * NC + lax.axis_index("c")
    base = wid * CPW

    pltpu.sync_copy(x_hbm.at[:, pl.ds(base, CPW)], xv)

    def amax_step(p, carry):
        cur_max, cur_idx = carry
        v = xv[p]
        pred = v > cur_max
        return (jnp.where(pred, v, cur_max), jnp.where(pred, p, cur_idx))

    init = (jnp.full((L,), -jnp.inf, jnp.float32), jnp.zeros((L,), jnp.int32))
    _, mu = lax.fori_loop(0, P, amax_step, init)
    img_v = jnp.full((L,), IMG, jnp.int32)
    row_i = lax.div(mu, img_v)
    col_i = lax.rem(mu, img_v)
    row = row_i.astype(jnp.float32)
    col = col_i.astype(jnp.float32)

    def row_loop(i, _):
        i_v = jnp.full((L,), i, jnp.int32).astype(jnp.float32)
        di = jnp.abs(i_v - row)

        def col_loop(j, _):
            j_v = jnp.full((L,), j, jnp.int32).astype(jnp.float32)
            dist = di + jnp.abs(j_v - col)
            m = TAU * jnp.maximum(1.0 - (BETA / IMG) * dist, -1.0)
            p = i * IMG + j
            outv[p] = xv[p] * m
            zv[p] = jnp.zeros((L,), jnp.float32)
            return 0

        lax.fori_loop(0, IMG, col_loop, 0)
        return 0

    lax.fori_loop(0, IMG, row_loop, 0)

    pltpu.sync_copy(outv, out_hbm.at[0, :, pl.ds(base, CPW)])
    for b in range(1, B):
        pltpu.sync_copy(zv, out_hbm.at[b, :, pl.ds(base, CPW)])


_sc_call = pl.kernel(
    _sc_body,
    out_type=jax.ShapeDtypeStruct((B, P, D), jnp.float32),
    mesh=plsc.VectorSubcoreMesh(core_axis_name="c", subcore_axis_name="s"),
    scratch_types=[
        pltpu.VMEM((P, L), jnp.float32),
        pltpu.VMEM((P, L), jnp.float32),
        pltpu.VMEM((P, L), jnp.float32),
    ],
    compiler_params=pltpu.CompilerParams(use_tc_tiling_on_sc=False),
)


@jax.jit
def kernel(x):
    x0 = x[0].reshape(P, D)
    out = _sc_call(x0)
    return out.reshape(B, IMG, IMG, D)
